# baseline (speedup 1.0000x reference)
import functools

import jax
import jax.numpy as jnp
from jax import lax
from jax.experimental import pallas as pl
from jax.experimental.pallas import tpu as pltpu
from jax.experimental.pallas import tpu_sc as plsc

_NC, _NS = 2, 16


def _sc_body(etop_hbm, ebot_hbm, out_hbm, top_v, bot_v, buf, shared, sem):
    f2 = out_hbm.shape[1]
    bsz = out_hbm.shape[0]
    half = f2 // 2
    nch = half // _NS
    cid = lax.axis_index("c")
    sid = lax.axis_index("s")
    c0 = sid * nch

    bch = buf.shape[0]

    @pl.when(cid == 0)
    def _():
        pltpu.sync_copy(etop_hbm.at[pl.ds(c0, nch)], top_v)
        for chunk in range(nch // bch):
            for cl in range(bch):
                vs = [top_v[chunk * bch + cl, pl.ds(16 * j, 16)] for j in range(8)]

                def qq_body(qq, carry, cl=cl, vs=vs):
                    for j in range(8):
                        buf[cl, qq, pl.ds(16 * j, 16)] = vs[j]
                    return carry

                lax.fori_loop(0, 32, qq_body, 0)
            pltpu.sync_copy(buf, shared.at[pl.ds(c0 + chunk * bch, bch)])

    @pl.when(cid == 1)
    def _():
        pltpu.sync_copy(ebot_hbm.at[pl.ds(c0, nch)], bot_v)
        for chunk in range(nch // bch):
            for cl in range(bch):

                def qq_body(qq, carry, cl=cl, chunk=chunk):
                    for j in range(2):
                        v = bot_v[chunk * bch + cl, pl.ds((2 * qq + j) * 16, 16)]
                        for k in range(4):
                            buf[cl, qq, pl.ds(j * 64 + k * 16, 16)] = v
                    return carry

                lax.fori_loop(0, 32, qq_body, 0)
            pltpu.sync_copy(buf, shared.at[pl.ds(c0 + chunk * bch, bch)])

    plsc.subcore_barrier()

    cp = pltpu.async_copy(shared, out_hbm.at[sid, pl.ds(cid * half, half)], sem)
    cp.wait()


def kernel(x, row_embed, col_embed):
    bsz, _, h, w = x.shape
    f = row_embed.shape[1]
    nch = f // _NS
    ct = col_embed[:w, :].T
    rt = row_embed[:h, :].T
    etop = jnp.concatenate([ct, ct], axis=1)
    ebot = jnp.broadcast_to(rt[:, :, None], (f, h, 16)).reshape(f, h * 16)

    mesh = plsc.VectorSubcoreMesh(
        core_axis_name="c", subcore_axis_name="s",
        num_cores=_NC, num_subcores=_NS,
    )
    run = functools.partial(
        pl.kernel,
        out_type=jax.ShapeDtypeStruct((bsz, 2 * f, h * w // 128, 128), jnp.float32),
        mesh=mesh,
        scratch_types=[
            pltpu.VMEM((nch, 128), jnp.float32),
            pltpu.VMEM((nch, h * 16), jnp.float32),
            pltpu.VMEM((nch // 4, h * w // 128, 128), jnp.float32),
            pltpu.MemorySpace.VMEM_SHARED((f, h * w // 128, 128), jnp.float32),
            pltpu.SemaphoreType.DMA,
        ],
    )(_sc_body)
    out = run(etop, ebot)
    return out.reshape(bsz, 2 * f, h, w)


# speedup vs baseline: 1.2245x; 1.2245x over previous
import jax
import jax.numpy as jnp
from jax.experimental import pallas as pl
from jax.experimental.pallas import tpu as pltpu


def _body(etop_ref, eb_ref, out_ref):
    f, q = eb_ref.shape[1], out_ref.shape[2]
    out_ref[0, :f] = jnp.broadcast_to(etop_ref[...][:, None, :], (f, q, 128))
    out_ref[0, f:, :, :64] = jnp.broadcast_to(eb_ref[0][:, :, None], (f, q, 64))
    out_ref[0, f:, :, 64:] = jnp.broadcast_to(eb_ref[1][:, :, None], (f, q, 64))


def kernel(x, row_embed, col_embed):
    bsz, _, h, w = x.shape
    f = row_embed.shape[1]
    ct = col_embed[:w, :].T
    rt = row_embed[:h, :].T
    etop = jnp.concatenate([ct, ct], axis=1)
    ebot = jnp.stack([rt[:, 0::2], rt[:, 1::2]])
    out = pl.pallas_call(
        _body,
        grid=(bsz,),
        in_specs=[
            pl.BlockSpec((f, 128), lambda b: (0, 0)),
            pl.BlockSpec((2, f, h // 2), lambda b: (0, 0, 0)),
        ],
        out_specs=pl.BlockSpec((1, 2 * f, h * w // 128, 128), lambda b: (b, 0, 0, 0)),
        out_shape=jax.ShapeDtypeStruct((bsz, 2 * f, h * w // 128, 128), jnp.float32),
    )(etop, ebot)
    return out.reshape(bsz, 2 * f, h, w)
